# baseline (speedup 1.0000x reference)
import jax
import jax.numpy as jnp
from jax.experimental import pallas as pl


def _probe_kernel(dt_ref, rows_ref, out_ref):
    out_ref[...] = dt_ref[...] * 2.0 + jnp.sum(rows_ref[...])


def kernel(x, delta_t, k, tables, W1, b1, a1, W2, b2, a2, W3, b3, a3,
           W4, b4, a4, W5, b5):
    B = delta_t.shape[0]
    rows = jnp.concatenate([t[0:1] for t in tables]
                           + [t[1:2] for t in tables], axis=0)
    dt2 = delta_t[None, :]
    out = pl.pallas_call(
        _probe_kernel,
        grid=(1,),
        in_specs=[pl.BlockSpec((1, B), lambda i: (0, 0)),
                  pl.BlockSpec((44, 16), lambda i: (0, 0))],
        out_specs=pl.BlockSpec((1, B), lambda i: (0, 0)),
        out_shape=jax.ShapeDtypeStruct((1, B), jnp.float32),
    )(dt2, rows)
    return out.reshape(B, 1)
